# baseline (speedup 1.0000x reference)
import functools

import jax
import jax.numpy as jnp
from jax import lax
from jax.experimental import pallas as pl
from jax.experimental.pallas import tpu as pltpu
from jax.experimental.pallas import tpu_sc as plsc

_ROWS = 1000000
_DIM = 64
_TOTAL = 4096 * 50
_NW = 32
_PER_W = _TOTAL // _NW
_CHUNK = 800
_NCHUNK = _PER_W // _CHUNK

_W = 256
_NGRP = _ROWS // _W
_G = _NGRP // _NW // 2
_NMAIN = _NW * 2 * _G

_mesh = plsc.VectorSubcoreMesh(core_axis_name="c", subcore_axis_name="s")


def _transpose_block(chunk_ref, tchunk_ref, width):
    iot = lax.iota(jnp.int32, 16)

    @plsc.parallel_loop(0, width, step=1, unroll=8)
    def _(r):
        rv = jnp.full((16,), r, jnp.int32)
        for k in range(4):
            v = plsc.load_gather(chunk_ref, [iot + 16 * k, rv])
            tchunk_ref[r, pl.ds(16 * k, 16)] = v


@functools.partial(
    pl.kernel,
    mesh=_mesh,
    out_type=jax.ShapeDtypeStruct((_ROWS, _DIM), jnp.float32),
    compiler_params=pltpu.CompilerParams(needs_layout_passes=False),
    scratch_types=[
        pltpu.VMEM((2, _DIM, _W + 1), jnp.float32),
        pltpu.VMEM((2, _W, _DIM), jnp.float32),
        pltpu.SemaphoreType.DMA,
        pltpu.SemaphoreType.DMA,
        pltpu.SemaphoreType.DMA,
        pltpu.SemaphoreType.DMA,
    ],
)
def _transpose_k(tT, tail, table_r, chunk, tchunk, i0, i1, o0, o1):
    isem = (i0, i1)
    osem = (o0, o1)
    wid = lax.axis_index("s") * 2 + lax.axis_index("c")

    def start_in(p, t):
        pltpu.async_copy(
            tT.at[:, pl.ds(t * _W, _W)], chunk.at[p, :, pl.ds(0, _W)], isem[p]
        )

    def wait_in(p, t):
        pltpu.make_async_copy(
            tT.at[:, pl.ds(t * _W, _W)], chunk.at[p, :, pl.ds(0, _W)], isem[p]
        ).wait()

    def start_out(p, t):
        pltpu.async_copy(
            tchunk.at[p], table_r.at[pl.ds(t * _W, _W), :], osem[p]
        )

    def wait_out(p, t):
        pltpu.make_async_copy(
            tchunk.at[p], table_r.at[pl.ds(t * _W, _W), :], osem[p]
        ).wait()

    start_in(0, wid)
    start_in(1, wid + _NW)

    def body(g, carry):
        q0 = 2 * g
        for p in range(2):
            q = q0 + p
            t = wid + _NW * q
            wait_in(p, t)

            @pl.when(g > 0)
            def _():
                wait_out(p, t - 2 * _NW)

            _transpose_block(chunk.at[p], tchunk.at[p], _W)
            start_out(p, t)

            @pl.when(g < _G - 1)
            def _():
                start_in(p, t + 2 * _NW)

        return carry

    lax.fori_loop(0, _G, body, 0)
    wait_out(0, wid + _NW * (2 * _G - 2))
    wait_out(1, wid + _NW * (2 * _G - 1))

    @pl.when(wid < _NGRP - _NMAIN)
    def _():
        t = _NMAIN + wid
        pltpu.sync_copy(
            tT.at[:, pl.ds(t * _W, _W)], chunk.at[0, :, pl.ds(0, _W)]
        )
        _transpose_block(chunk.at[0], tchunk.at[0], _W)
        pltpu.sync_copy(tchunk.at[0], table_r.at[pl.ds(t * _W, _W), :])

    @pl.when(wid == _NGRP - _NMAIN)
    def _():
        base = _NGRP * _W
        pltpu.sync_copy(tail, tchunk.at[1, pl.ds(0, _ROWS - base), :])
        pltpu.sync_copy(
            tchunk.at[1, pl.ds(0, _ROWS - base), :],
            table_r.at[pl.ds(base, _ROWS - base), :],
        )


_B = 4096
_H = 50


@functools.partial(
    pl.kernel,
    mesh=_mesh,
    out_type=jax.ShapeDtypeStruct((_H, 8, _NW, 1024), jnp.float32),
    compiler_params=pltpu.CompilerParams(
        use_tc_tiling_on_sc=False, needs_layout_passes=False
    ),
    scratch_types=[
        pltpu.VMEM((_H, 128), jnp.int32),
        pltpu.VMEM((8, 128, _DIM), jnp.float32),
        pltpu.VMEM((2, 8, 1024), jnp.float32),
        pltpu.SemaphoreType.DMA,
        pltpu.SemaphoreType.DMA,
        pltpu.SemaphoreType.DMA,
        pltpu.SemaphoreType.DMA,
        pltpu.SemaphoreType.DMA,
        pltpu.SemaphoreType.DMA,
        pltpu.SemaphoreType.DMA,
        pltpu.SemaphoreType.DMA,
        pltpu.SemaphoreType.DMA,
        pltpu.SemaphoreType.DMA,
    ],
)
def _gather_fused(
    idx_hbm, table_hbm, out_hbm, idx_v, rows_v, pan_v,
    g0, g1, g2, g3, g4, g5, g6, g7, s0, s1,
):
    gsem = (g0, g1, g2, g3, g4, g5, g6, g7)
    ssem = (s0, s1)
    wid = lax.axis_index("s") * 2 + lax.axis_index("c")
    pltpu.sync_copy(idx_hbm.at[:, pl.ds(wid * 128, 128)], idx_v)
    iot = lax.iota(jnp.int32, 16)

    def start_gather(p, h):
        pltpu.async_copy(
            table_hbm.at[idx_v.at[h]], rows_v.at[p], gsem[p]
        )

    def wait_gather(p, h):
        pltpu.make_async_copy(
            table_hbm.at[idx_v.at[h]], rows_v.at[p], gsem[p]
        ).wait()

    def start_store(p, h):
        pltpu.async_copy(
            pan_v.at[p], out_hbm.at[h, :, wid, :], ssem[p]
        )

    def wait_store(p, h):
        pltpu.make_async_copy(
            pan_v.at[p], out_hbm.at[h, :, wid, :], ssem[p]
        ).wait()

    def transpose_chunk(p, p2):
        rows = rows_v.at[p]
        pan = pan_v.at[p2]

        @plsc.parallel_loop(0, _DIM, step=1, unroll=8)
        def _(c):
            cv = jnp.full((16,), c, jnp.int32)
            for k in range(8):
                v = plsc.load_gather(rows, [iot + 16 * k, cv])
                pan[c // 8, pl.ds((c % 8) * 128 + 16 * k, 16)] = v

    for j in range(6):
        start_gather(j, j)

    def body(G, carry):
        for j in range(8):
            h = 8 * G + j
            p2 = j % 2
            wait_gather(j, h)

            @pl.when(h >= 2)
            def _():
                wait_store(p2, h - 2)

            transpose_chunk(j, p2)
            start_store(p2, h)

            @pl.when(h + 6 < _H)
            def _():
                start_gather((j + 6) % 8, h + 6)

        return carry

    lax.fori_loop(0, _H // 8, body, 0)
    for h in range(_H - _H % 8, _H):
        j = h % 8
        p2 = j % 2
        wait_gather(j, h)
        wait_store(p2, h - 2)
        transpose_chunk(j, p2)
        start_store(p2, h)
    wait_store(0, _H - 2)
    wait_store(1, _H - 1)


@functools.partial(
    pl.kernel,
    mesh=_mesh,
    out_type=jax.ShapeDtypeStruct((_TOTAL, _DIM), jnp.float32),
    compiler_params=pltpu.CompilerParams(use_tc_tiling_on_sc=False),
    scratch_types=[
        pltpu.VMEM((_PER_W,), jnp.int32),
        pltpu.VMEM((2, _CHUNK, _DIM), jnp.float32),
        pltpu.SemaphoreType.DMA,
        pltpu.SemaphoreType.DMA,
        pltpu.SemaphoreType.DMA,
        pltpu.SemaphoreType.DMA,
    ],
)
def _gather(idx_hbm, table_hbm, out_hbm, idx_v, rows_v, g0, g1, s0, s1):
    gsem = (g0, g1)
    ssem = (s0, s1)
    wid = lax.axis_index("s") * 2 + lax.axis_index("c")
    base = wid * _PER_W
    pltpu.sync_copy(idx_hbm.at[pl.ds(base, _PER_W)], idx_v)

    def start_gather(c):
        b = c % 2
        pltpu.async_copy(
            table_hbm.at[idx_v.at[pl.ds(c * _CHUNK, _CHUNK)]],
            rows_v.at[b],
            gsem[b],
        )

    def wait_gather(c):
        b = c % 2
        pltpu.make_async_copy(
            table_hbm.at[idx_v.at[pl.ds(c * _CHUNK, _CHUNK)]],
            rows_v.at[b],
            gsem[b],
        ).wait()

    def start_store(c):
        b = c % 2
        pltpu.async_copy(
            rows_v.at[b], out_hbm.at[pl.ds(base + c * _CHUNK, _CHUNK)], ssem[b]
        )

    def wait_store(c):
        b = c % 2
        pltpu.make_async_copy(
            rows_v.at[b], out_hbm.at[pl.ds(base + c * _CHUNK, _CHUNK)], ssem[b]
        ).wait()

    start_gather(0)
    for c in range(1, _NCHUNK):
        if c >= 2:
            wait_store(c - 2)
        start_gather(c)
        wait_gather(c - 1)
        start_store(c - 1)
    wait_gather(_NCHUNK - 1)
    start_store(_NCHUNK - 1)
    wait_store(_NCHUNK - 2)
    wait_store(_NCHUNK - 1)


def kernel(model_input, table):
    idxT = model_input.T.astype(jnp.int32)
    out4 = _gather_fused(idxT, table)
    t5 = out4.reshape(_H, 8, _NW, 8, 128)
    return t5.transpose(2, 4, 0, 1, 3).reshape(_B, _H, _DIM)


# speedup vs baseline: 1.0279x; 1.0279x over previous
import functools

import jax
import jax.numpy as jnp
from jax import lax
from jax.experimental import pallas as pl
from jax.experimental.pallas import tpu as pltpu
from jax.experimental.pallas import tpu_sc as plsc

_ROWS = 1000000
_DIM = 64
_TOTAL = 4096 * 50
_NW = 32
_PER_W = _TOTAL // _NW
_CHUNK = 800
_NCHUNK = _PER_W // _CHUNK

_W = 256
_NGRP = _ROWS // _W
_G = _NGRP // _NW // 2
_NMAIN = _NW * 2 * _G

_mesh = plsc.VectorSubcoreMesh(core_axis_name="c", subcore_axis_name="s")


def _transpose_block(chunk_ref, tchunk_ref, width):
    iot = lax.iota(jnp.int32, 16)

    @plsc.parallel_loop(0, width, step=1, unroll=8)
    def _(r):
        rv = jnp.full((16,), r, jnp.int32)
        for k in range(4):
            v = plsc.load_gather(chunk_ref, [iot + 16 * k, rv])
            tchunk_ref[r, pl.ds(16 * k, 16)] = v


@functools.partial(
    pl.kernel,
    mesh=_mesh,
    out_type=jax.ShapeDtypeStruct((_ROWS, _DIM), jnp.float32),
    compiler_params=pltpu.CompilerParams(needs_layout_passes=False),
    scratch_types=[
        pltpu.VMEM((2, _DIM, _W + 1), jnp.float32),
        pltpu.VMEM((2, _W, _DIM), jnp.float32),
        pltpu.SemaphoreType.DMA,
        pltpu.SemaphoreType.DMA,
        pltpu.SemaphoreType.DMA,
        pltpu.SemaphoreType.DMA,
    ],
)
def _transpose_k(tT, tail, table_r, chunk, tchunk, i0, i1, o0, o1):
    isem = (i0, i1)
    osem = (o0, o1)
    wid = lax.axis_index("s") * 2 + lax.axis_index("c")

    def start_in(p, t):
        pltpu.async_copy(
            tT.at[:, pl.ds(t * _W, _W)], chunk.at[p, :, pl.ds(0, _W)], isem[p]
        )

    def wait_in(p, t):
        pltpu.make_async_copy(
            tT.at[:, pl.ds(t * _W, _W)], chunk.at[p, :, pl.ds(0, _W)], isem[p]
        ).wait()

    def start_out(p, t):
        pltpu.async_copy(
            tchunk.at[p], table_r.at[pl.ds(t * _W, _W), :], osem[p]
        )

    def wait_out(p, t):
        pltpu.make_async_copy(
            tchunk.at[p], table_r.at[pl.ds(t * _W, _W), :], osem[p]
        ).wait()

    start_in(0, wid)
    start_in(1, wid + _NW)

    def body(g, carry):
        q0 = 2 * g
        for p in range(2):
            q = q0 + p
            t = wid + _NW * q
            wait_in(p, t)

            @pl.when(g > 0)
            def _():
                wait_out(p, t - 2 * _NW)

            _transpose_block(chunk.at[p], tchunk.at[p], _W)
            start_out(p, t)

            @pl.when(g < _G - 1)
            def _():
                start_in(p, t + 2 * _NW)

        return carry

    lax.fori_loop(0, _G, body, 0)
    wait_out(0, wid + _NW * (2 * _G - 2))
    wait_out(1, wid + _NW * (2 * _G - 1))

    @pl.when(wid < _NGRP - _NMAIN)
    def _():
        t = _NMAIN + wid
        pltpu.sync_copy(
            tT.at[:, pl.ds(t * _W, _W)], chunk.at[0, :, pl.ds(0, _W)]
        )
        _transpose_block(chunk.at[0], tchunk.at[0], _W)
        pltpu.sync_copy(tchunk.at[0], table_r.at[pl.ds(t * _W, _W), :])

    @pl.when(wid == _NGRP - _NMAIN)
    def _():
        base = _NGRP * _W
        pltpu.sync_copy(tail, tchunk.at[1, pl.ds(0, _ROWS - base), :])
        pltpu.sync_copy(
            tchunk.at[1, pl.ds(0, _ROWS - base), :],
            table_r.at[pl.ds(base, _ROWS - base), :],
        )


_B = 4096
_H = 50


@functools.partial(
    pl.kernel,
    mesh=_mesh,
    out_type=jax.ShapeDtypeStruct((_H, 8, _NW, 1024), jnp.float32),
    compiler_params=pltpu.CompilerParams(
        use_tc_tiling_on_sc=False, needs_layout_passes=False
    ),
    scratch_types=[
        pltpu.VMEM((_H, 128), jnp.int32),
        pltpu.VMEM((8, 128, _DIM), jnp.float32),
        pltpu.VMEM((2, 8, 1024), jnp.float32),
        pltpu.SemaphoreType.DMA,
        pltpu.SemaphoreType.DMA,
        pltpu.SemaphoreType.DMA,
        pltpu.SemaphoreType.DMA,
        pltpu.SemaphoreType.DMA,
        pltpu.SemaphoreType.DMA,
        pltpu.SemaphoreType.DMA,
        pltpu.SemaphoreType.DMA,
        pltpu.SemaphoreType.DMA,
        pltpu.SemaphoreType.DMA,
    ],
)
def _gather_fused(
    idx_hbm, table_hbm, out_hbm, idx_v, rows_v, pan_v,
    g0, g1, g2, g3, g4, g5, g6, g7, s0, s1,
):
    gsem = (g0, g1, g2, g3, g4, g5, g6, g7)
    ssem = (s0, s1)
    wid = lax.axis_index("s") * 2 + lax.axis_index("c")
    pltpu.sync_copy(idx_hbm.at[:, pl.ds(wid * 128, 128)], idx_v)
    iot = lax.iota(jnp.int32, 16)

    def start_gather(p, h):
        pltpu.async_copy(
            table_hbm.at[idx_v.at[h]], rows_v.at[p], gsem[p]
        )

    def wait_gather(p, h):
        pltpu.make_async_copy(
            table_hbm.at[idx_v.at[h]], rows_v.at[p], gsem[p]
        ).wait()

    def start_store(p, h):
        pltpu.async_copy(
            pan_v.at[p], out_hbm.at[h, :, wid, :], ssem[p]
        )

    def wait_store(p, h):
        pltpu.make_async_copy(
            pan_v.at[p], out_hbm.at[h, :, wid, :], ssem[p]
        ).wait()

    def transpose_chunk(p, p2):
        rows = rows_v.at[p]
        pan = pan_v.at[p2]

        @plsc.parallel_loop(0, _DIM, step=1, unroll=8)
        def _(c):
            cv = jnp.full((16,), c, jnp.int32)
            for k in range(8):
                v = plsc.load_gather(rows, [iot + 16 * k, cv])
                pan[c // 8, pl.ds((c % 8) * 128 + 16 * k, 16)] = v

    for j in range(6):
        start_gather(j, j)

    def body(G, carry):
        for j in range(8):
            h = 8 * G + j
            p2 = j % 2
            wait_gather(j, h)

            @pl.when(h >= 2)
            def _():
                wait_store(p2, h - 2)

            transpose_chunk(j, p2)
            start_store(p2, h)

            @pl.when(h + 6 < _H)
            def _():
                start_gather((j + 6) % 8, h + 6)

        return carry

    lax.fori_loop(0, _H // 8, body, 0)
    for h in range(_H - _H % 8, _H):
        j = h % 8
        p2 = j % 2
        wait_gather(j, h)
        wait_store(p2, h - 2)
        transpose_chunk(j, p2)
        start_store(p2, h)
    wait_store(0, _H - 2)
    wait_store(1, _H - 1)


@functools.partial(
    pl.kernel,
    mesh=_mesh,
    out_type=jax.ShapeDtypeStruct((_TOTAL, _DIM), jnp.float32),
    compiler_params=pltpu.CompilerParams(
        use_tc_tiling_on_sc=False, skip_device_barrier=True
    ),
    scratch_types=[
        pltpu.VMEM((_PER_W,), jnp.int32),
        pltpu.VMEM((2, _CHUNK, _DIM), jnp.float32),
        pltpu.SemaphoreType.DMA,
        pltpu.SemaphoreType.DMA,
        pltpu.SemaphoreType.DMA,
        pltpu.SemaphoreType.DMA,
    ],
)
def _gather(idx_hbm, table_hbm, out_hbm, idx_v, rows_v, g0, g1, s0, s1):
    gsem = (g0, g1)
    ssem = (s0, s1)
    wid = lax.axis_index("s") * 2 + lax.axis_index("c")
    base = wid * _PER_W
    pltpu.sync_copy(idx_hbm.at[pl.ds(base, _PER_W)], idx_v)

    def start_gather(c):
        b = c % 2
        pltpu.async_copy(
            table_hbm.at[idx_v.at[pl.ds(c * _CHUNK, _CHUNK)]],
            rows_v.at[b],
            gsem[b],
        )

    def wait_gather(c):
        b = c % 2
        pltpu.make_async_copy(
            table_hbm.at[idx_v.at[pl.ds(c * _CHUNK, _CHUNK)]],
            rows_v.at[b],
            gsem[b],
        ).wait()

    def start_store(c):
        b = c % 2
        pltpu.async_copy(
            rows_v.at[b], out_hbm.at[pl.ds(base + c * _CHUNK, _CHUNK)], ssem[b]
        )

    def wait_store(c):
        b = c % 2
        pltpu.make_async_copy(
            rows_v.at[b], out_hbm.at[pl.ds(base + c * _CHUNK, _CHUNK)], ssem[b]
        ).wait()

    start_gather(0)
    for c in range(1, _NCHUNK):
        if c >= 2:
            wait_store(c - 2)
        start_gather(c)
        wait_gather(c - 1)
        start_store(c - 1)
    wait_gather(_NCHUNK - 1)
    start_store(_NCHUNK - 1)
    wait_store(_NCHUNK - 2)
    wait_store(_NCHUNK - 1)


def kernel(model_input, table):
    idx = model_input.T.reshape(-1).astype(jnp.int32)
    out = _gather(idx, table)
    return out.reshape(_H, _B, _DIM).transpose(1, 0, 2)
